# baseline (speedup 1.0000x reference)
import functools

import jax
import jax.numpy as jnp
from jax import lax
from jax.experimental import pallas as pl
from jax.experimental.pallas import tpu as pltpu
from jax.experimental.pallas import tpu_sc as plsc

_ROWS = 1024
_SEQ = 26
_SEQ_PAD = 32
_NCLS = 1000
_NWORKERS = 16
_SLICES_PER_W = _ROWS // _NWORKERS
_IDX_PER_W = _SLICES_PER_W * _SEQ_PAD


_NBUF = 3


def _sc_onehot(data_hbm, tmpl_hbm, rvals_hbm, out_hbm, idx_v, rvals_v,
               *bufs_and_sems):
    bufs = bufs_and_sems[:_NBUF]
    sems = bufs_and_sems[_NBUF:]
    wid = lax.axis_index("s")
    base_slice = wid * _SLICES_PER_W

    pltpu.sync_copy(data_hbm.at[pl.ds(wid * _IDX_PER_W, _IDX_PER_W)], idx_v)
    pltpu.sync_copy(rvals_hbm.at[pl.ds(wid * _IDX_PER_W, _IDX_PER_W)], rvals_v)
    for buf in bufs:
        pltpu.sync_copy(tmpl_hbm, buf)

    riota = lax.iota(jnp.int32, 16)
    ones = jnp.full((16,), 1.0, jnp.float32)

    def poke(local_slice, buf):
        for g in range(2):
            cols = idx_v[pl.ds(local_slice * _SEQ_PAD + g * 16, 16)]
            rows = riota + g * 16
            mask = rows < _SEQ
            plsc.store_scatter(buf, [rows, cols], ones, mask=mask)

    def restore(local_slice, buf):
        for g in range(2):
            cols = idx_v[pl.ds(local_slice * _SEQ_PAD + g * 16, 16)]
            rows = riota + g * 16
            mask = rows < _SEQ
            vals = rvals_v[pl.ds(local_slice * _SEQ_PAD + g * 16, 16)]
            plsc.store_scatter(buf, [rows, cols], vals, mask=mask)

    def step(k, carry):
        for b in range(_NBUF):
            buf, sem = bufs[b], sems[b]
            i = _NBUF * k + b

            @pl.when(i < _SLICES_PER_W)
            def _do_slice():
                g = base_slice + i

                @pl.when(i >= _NBUF)
                def _wait_and_restore():
                    pltpu.make_async_copy(buf, out_hbm.at[g - _NBUF], sem).wait()
                    restore(i - _NBUF, buf)

                poke(i, buf)
                pltpu.async_copy(buf, out_hbm.at[g], sem)
        return carry

    n_iters = (_SLICES_PER_W + _NBUF - 1) // _NBUF
    lax.fori_loop(0, n_iters, step, 0)

    for i_last in range(_SLICES_PER_W - _NBUF, _SLICES_PER_W):
        b = i_last % _NBUF
        pltpu.make_async_copy(
            bufs[b], out_hbm.at[base_slice + i_last], sems[b]).wait()


@jax.jit
def _run(data_pad, tmpl, rvals):
    mesh = plsc.VectorSubcoreMesh(core_axis_name="c", subcore_axis_name="s", num_cores=1)
    return pl.kernel(
        _sc_onehot,
        mesh=mesh,
        out_type=jax.ShapeDtypeStruct((_ROWS, _SEQ, _NCLS), jnp.float32),
        scratch_types=[
            pltpu.VMEM((_IDX_PER_W,), jnp.int32),
            pltpu.VMEM((_IDX_PER_W,), jnp.float32),
        ] + [pltpu.VMEM((_SEQ, _NCLS), jnp.float32)] * _NBUF
          + [pltpu.SemaphoreType.DMA] * _NBUF,
        compiler_params=pltpu.CompilerParams(needs_layout_passes=False),
    )(data_pad, tmpl, rvals)


def kernel(input, one_hot):
    data_pad = jnp.pad(input.astype(jnp.int32), ((0, 0), (0, _SEQ_PAD - _SEQ)))
    tmpl = jnp.tile(one_hot.astype(jnp.float32), (_SEQ, 1))
    flat = data_pad.reshape(-1)
    rvals = jnp.take(one_hot.astype(jnp.float32)[0], flat)
    return _run(flat, tmpl, rvals)


# speedup vs baseline: 1.0517x; 1.0517x over previous
import functools

import jax
import jax.numpy as jnp
from jax import lax
from jax.experimental import pallas as pl
from jax.experimental.pallas import tpu as pltpu
from jax.experimental.pallas import tpu_sc as plsc

_ROWS = 1024
_SEQ = 26
_SEQ_PAD = 32
_NCLS = 1000
_NWORKERS = 32
_SLICES_PER_W = _ROWS // _NWORKERS
_IDX_PER_W = _SLICES_PER_W * _SEQ_PAD


_NBUF = 3


def _sc_onehot(data_hbm, tmpl_hbm, rvals_hbm, out_hbm, idx_v, rvals_v,
               *bufs_and_sems):
    bufs = bufs_and_sems[:_NBUF]
    sems = bufs_and_sems[_NBUF:]
    wid = lax.axis_index("s") * 2 + lax.axis_index("c")
    base_slice = wid * _SLICES_PER_W

    pltpu.sync_copy(data_hbm.at[pl.ds(wid * _IDX_PER_W, _IDX_PER_W)], idx_v)
    pltpu.sync_copy(rvals_hbm.at[pl.ds(wid * _IDX_PER_W, _IDX_PER_W)], rvals_v)
    for buf in bufs:
        pltpu.sync_copy(tmpl_hbm, buf)

    riota = lax.iota(jnp.int32, 16)
    ones = jnp.full((16,), 1.0, jnp.float32)

    def poke(local_slice, buf):
        for g in range(2):
            cols = idx_v[pl.ds(local_slice * _SEQ_PAD + g * 16, 16)]
            rows = riota + g * 16
            mask = rows < _SEQ
            plsc.store_scatter(buf, [rows, cols], ones, mask=mask)

    def restore(local_slice, buf):
        for g in range(2):
            cols = idx_v[pl.ds(local_slice * _SEQ_PAD + g * 16, 16)]
            rows = riota + g * 16
            mask = rows < _SEQ
            vals = rvals_v[pl.ds(local_slice * _SEQ_PAD + g * 16, 16)]
            plsc.store_scatter(buf, [rows, cols], vals, mask=mask)

    def step(k, carry):
        for b in range(_NBUF):
            buf, sem = bufs[b], sems[b]
            i = _NBUF * k + b

            @pl.when(i < _SLICES_PER_W)
            def _do_slice():
                g = base_slice + i

                @pl.when(i >= _NBUF)
                def _wait_and_restore():
                    pltpu.make_async_copy(buf, out_hbm.at[g - _NBUF], sem).wait()
                    restore(i - _NBUF, buf)

                poke(i, buf)
                pltpu.async_copy(buf, out_hbm.at[g], sem)
        return carry

    n_iters = (_SLICES_PER_W + _NBUF - 1) // _NBUF
    lax.fori_loop(0, n_iters, step, 0)

    for i_last in range(_SLICES_PER_W - _NBUF, _SLICES_PER_W):
        b = i_last % _NBUF
        pltpu.make_async_copy(
            bufs[b], out_hbm.at[base_slice + i_last], sems[b]).wait()


@jax.jit
def _run(data_pad, tmpl, rvals):
    mesh = plsc.VectorSubcoreMesh(core_axis_name="c", subcore_axis_name="s")
    return pl.kernel(
        _sc_onehot,
        mesh=mesh,
        out_type=jax.ShapeDtypeStruct((_ROWS, _SEQ, _NCLS), jnp.float32),
        scratch_types=[
            pltpu.VMEM((_IDX_PER_W,), jnp.int32),
            pltpu.VMEM((_IDX_PER_W,), jnp.float32),
        ] + [pltpu.VMEM((_SEQ, _NCLS), jnp.float32)] * _NBUF
          + [pltpu.SemaphoreType.DMA] * _NBUF,
        compiler_params=pltpu.CompilerParams(
            needs_layout_passes=False, skip_device_barrier=True),
    )(data_pad, tmpl, rvals)


def kernel(input, one_hot):
    data_pad = jnp.pad(input.astype(jnp.int32), ((0, 0), (0, _SEQ_PAD - _SEQ)))
    tmpl = jnp.tile(one_hot.astype(jnp.float32), (_SEQ, 1))
    flat = data_pad.reshape(-1)
    rvals = jnp.take(one_hot.astype(jnp.float32)[0], flat)
    return _run(flat, tmpl, rvals)


# speedup vs baseline: 2.6552x; 2.5246x over previous
import jax
import jax.numpy as jnp
from jax.experimental import pallas as pl
from jax.experimental.pallas import tpu as pltpu

_ROWS_PER_BLOCK = 64


def _onehot_block(idx_ref, oh_ref, out_ref):
    idx = idx_ref[...]
    b, s, ncls = out_ref.shape
    iota = jax.lax.broadcasted_iota(jnp.int32, (b, s, ncls), 2)
    base = oh_ref[0, :]
    out_ref[...] = jnp.where(iota == idx[:, :, None], 1.0, base)


def kernel(input, one_hot):
    rows, seq = input.shape
    ncls = one_hot.shape[-1]
    data = input.astype(jnp.int32)
    b = _ROWS_PER_BLOCK
    nb = rows // b
    out = pl.pallas_call(
        _onehot_block,
        grid=(nb,),
        in_specs=[
            pl.BlockSpec((b, seq), lambda i: (i, 0)),
            pl.BlockSpec((1, ncls), lambda i: (0, 0)),
        ],
        out_specs=pl.BlockSpec((b, seq, ncls), lambda i: (i, 0, 0)),
        out_shape=jax.ShapeDtypeStruct((rows, seq, ncls), jnp.float32),
    )(data, one_hot)
    return out


# speedup vs baseline: 2.6585x; 1.0012x over previous
import jax
import jax.numpy as jnp
from jax.experimental import pallas as pl

_ROWS_PER_BLOCK = 64


def _onehot_block(idx_ref, oh_ref, out_ref):
    idx = idx_ref[...]
    b, s, ncls = out_ref.shape
    iota = jax.lax.broadcasted_iota(jnp.int32, (b, s, ncls), 2)
    base = oh_ref[0, :]
    out_ref[...] = jnp.where(iota == idx[:, :, None], 1.0, base)


def kernel(input, one_hot):
    rows, seq = input.shape
    ncls = one_hot.shape[-1]
    data = input.astype(jnp.int32)
    b = _ROWS_PER_BLOCK
    nb = rows // b
    out = pl.pallas_call(
        _onehot_block,
        grid=(nb,),
        in_specs=[
            pl.BlockSpec((b, seq), lambda i: (i, 0)),
            pl.BlockSpec((1, ncls), lambda i: (0, 0)),
        ],
        out_specs=pl.BlockSpec((b, seq, ncls), lambda i: (i, 0, 0)),
        out_shape=jax.ShapeDtypeStruct((rows, seq, ncls), jnp.float32),
    )(data, one_hot)
    return out
